# baseline (speedup 1.0000x reference)
import functools

import jax
import jax.numpy as jnp
from jax import lax
from jax.experimental import pallas as pl
from jax.experimental.pallas import tpu as pltpu
from jax.experimental.pallas import tpu_sc as plsc

NC, NS = 2, 16
NW = NC * NS
NB_SAMPLES = 16384
S = 50
D = 32
SPW = NB_SAMPLES // NW
IPW = SPW * S

_mesh = plsc.VectorSubcoreMesh(core_axis_name="c", subcore_axis_name="s")


@functools.partial(
    pl.kernel,
    out_type=jax.ShapeDtypeStruct((S, D, NB_SAMPLES), jnp.float32),
    mesh=_mesh,
    scratch_types=[
        pltpu.VMEM((SPW, 64), jnp.int32),
        pltpu.VMEM((IPW,), jnp.int32),
        pltpu.VMEM((2, SPW, D), jnp.float32),
        pltpu.VMEM((2, D * SPW), jnp.float32),
        pltpu.SemaphoreType.DMA,
        pltpu.SemaphoreType.DMA,
        pltpu.SemaphoreType.DMA,
        pltpu.SemaphoreType.DMA,
    ],
    compiler_params=pltpu.CompilerParams(
        use_tc_tiling_on_sc=False, needs_layout_passes=False),
)
def _gather_kernel(idx_hbm, table_hbm, out_hbm, idx_v, idxt_v, g_v, t_v,
                   g0, g1, s0, s1):
    wid = lax.axis_index("s") * NC + lax.axis_index("c")
    b0 = wid * SPW
    gsem = (g0, g1)
    ssem = (s0, s1)

    pltpu.sync_copy(idx_hbm.at[pl.ds(b0, SPW), pl.ds(0, 64)], idx_v)

    lanes = lax.iota(jnp.int32, 16)
    scat_lo = lanes * SPW
    scat_hi = (lanes + 16) * SPW

    @pl.loop(0, S)
    def _slot(s):
        svec = jnp.full((16,), 0, jnp.int32) + s

        @plsc.parallel_loop(0, SPW // 16)
        def _grp(g):
            rvec = lanes + g * 16
            idxt_v[pl.ds(s * SPW + g * 16, 16)] = plsc.load_gather(
                idx_v, [rvec, svec])

    def fire_gather(s, b):
        pltpu.async_copy(
            table_hbm.at[idxt_v.at[pl.ds(s * SPW, SPW)]], g_v.at[b], gsem[b])

    def wait_gather(s, b):
        pltpu.make_async_copy(
            table_hbm.at[idxt_v.at[pl.ds(s * SPW, SPW)]], g_v.at[b], gsem[b]).wait()

    def drain_stores(b):
        for d in range(D):
            pltpu.make_async_copy(
                t_v.at[b, pl.ds(d * SPW, SPW)],
                out_hbm.at[0, d, pl.ds(b0, SPW)], ssem[b]).wait()

    fire_gather(0, 0)

    @pl.loop(0, S, step=2)
    def _pipe(si):
        for b in range(2):
            s = si + b
            wait_gather(s, b)

            @pl.when(s + 1 < S)
            def _next():
                fire_gather(s + 1, 1 - b)

            @pl.when(s >= 2)
            def _drain():
                drain_stores(b)

            @plsc.parallel_loop(0, SPW, unroll=8)
            def _row(r):
                plsc.store_scatter(t_v.at[b], [scat_lo + r],
                                   g_v[b, r, pl.ds(0, 16)])
                plsc.store_scatter(t_v.at[b], [scat_hi + r],
                                   g_v[b, r, pl.ds(16, 16)])

            for d in range(D):
                pltpu.async_copy(
                    t_v.at[b, pl.ds(d * SPW, SPW)],
                    out_hbm.at[s, d, pl.ds(b0, SPW)], ssem[b])

    for b in range(2):
        drain_stores(b)


def kernel(input, item_embedding):
    idxp = jnp.pad(input.astype(jnp.int32), ((0, 0), (0, 128 - S)))
    out = _gather_kernel(idxp, item_embedding)
    return jnp.transpose(out, (2, 0, 1))


# speedup vs baseline: 1.1426x; 1.1426x over previous
import functools

import jax
import jax.numpy as jnp
from jax import lax
from jax.experimental import pallas as pl
from jax.experimental.pallas import tpu as pltpu
from jax.experimental.pallas import tpu_sc as plsc

NC, NS = 2, 16
NW = NC * NS
NB_SAMPLES = 16384
S = 50
D = 32
SPW = NB_SAMPLES // NW
IPW = SPW * S
SLOT_W = D * NB_SAMPLES
TILE_ROW_W = 8 * NB_SAMPLES

_mesh = plsc.VectorSubcoreMesh(core_axis_name="c", subcore_axis_name="s")


@functools.partial(
    pl.kernel,
    out_type=jax.ShapeDtypeStruct((S * D * NB_SAMPLES,), jnp.float32),
    mesh=_mesh,
    scratch_types=[
        pltpu.VMEM((SPW, S), jnp.int32),
        pltpu.VMEM((IPW,), jnp.int32),
        pltpu.VMEM((2, SPW, D), jnp.float32),
        pltpu.VMEM((2, D * SPW), jnp.float32),
        pltpu.SemaphoreType.DMA,
        pltpu.SemaphoreType.DMA,
        pltpu.SemaphoreType.DMA,
        pltpu.SemaphoreType.DMA,
    ],
    compiler_params=pltpu.CompilerParams(
        use_tc_tiling_on_sc=False, needs_layout_passes=False),
)
def _gather_kernel(idx_hbm, table_hbm, out_hbm, idx_v, idxt_v, g_v, t_v,
                   g0, g1, s0, s1):
    wid = lax.axis_index("s") * NC + lax.axis_index("c")
    b0 = wid * SPW
    gsem = (g0, g1)
    ssem = (s0, s1)

    pltpu.sync_copy(idx_hbm.at[pl.ds(b0, SPW), :], idx_v)

    lanes = lax.iota(jnp.int32, 16)
    p_lo = (lanes // 8) * 4096 + (lanes % 8) * 128
    p_hi = p_lo + 8192

    @pl.loop(0, S)
    def _slot(s):
        svec = jnp.full((16,), 0, jnp.int32) + s

        @plsc.parallel_loop(0, SPW // 16)
        def _grp(g):
            rvec = lanes + g * 16
            idxt_v[pl.ds(s * SPW + g * 16, 16)] = plsc.load_gather(
                idx_v, [rvec, svec])

    def fire_gather(s, b):
        pltpu.async_copy(
            table_hbm.at[idxt_v.at[pl.ds(s * SPW, SPW)]], g_v.at[b], gsem[b])

    def wait_gather(s, b):
        pltpu.make_async_copy(
            table_hbm.at[idxt_v.at[pl.ds(s * SPW, SPW)]], g_v.at[b], gsem[b]).wait()

    def drain_stores(b):
        for ti in range(4):
            pltpu.make_async_copy(
                t_v.at[b, pl.ds(ti * 4096, 4096)],
                out_hbm.at[pl.ds(0, 4096)], ssem[b]).wait()

    fire_gather(0, 0)

    @pl.loop(0, S, step=2)
    def _pipe(si):
        for b in range(2):
            s = si + b
            wait_gather(s, b)

            @pl.when(s + 1 < S)
            def _next():
                fire_gather(s + 1, 1 - b)

            @pl.when(s >= 2)
            def _drain():
                drain_stores(b)

            @plsc.parallel_loop(0, SPW, unroll=8)
            def _row(r):
                base = (r // 128) * 1024 + (r % 128)
                plsc.store_scatter(t_v.at[b], [p_lo + base],
                                   g_v[b, r, pl.ds(0, 16)])
                plsc.store_scatter(t_v.at[b], [p_hi + base],
                                   g_v[b, r, pl.ds(16, 16)])

            for ti in range(4):
                pltpu.async_copy(
                    t_v.at[b, pl.ds(ti * 4096, 4096)],
                    out_hbm.at[pl.ds(s * SLOT_W + ti * TILE_ROW_W
                                     + wid * 4096, 4096)],
                    ssem[b])

    for b in range(2):
        drain_stores(b)


def kernel(input, item_embedding):
    out1d = _gather_kernel(input, item_embedding)
    v = out1d.reshape(S, 4, 128, 8, 128)
    return v.transpose(2, 4, 0, 1, 3).reshape(NB_SAMPLES, S, D)
